# baseline (device time: 98552 ns/iter reference)
import jax
import jax.numpy as jnp
from jax import lax
from jax.experimental import pallas as pl
from jax.experimental.pallas import tpu as pltpu

N_DEV = 16
N_SUB = 2


def kernel(x, w_mat, scale_x, scale_w):
    m, _ = x.shape
    _, n = w_mat.shape
    m_chunk = m // N_DEV
    n_half = n // 2

    sx = scale_x.reshape(1).astype(jnp.float32)
    sw = scale_w.reshape(1).astype(jnp.float32)

    def body(
        x_ref,
        w_ref,
        sx_ref,
        sw_ref,
        out_ref,
        comm_f,
        comm_b,
        send_f,
        recv_f,
        send_b,
        recv_b,
    ):
        d = lax.axis_index("i")
        right = lax.rem(d + 1, N_DEV)
        left = lax.rem(d - 1 + N_DEV, N_DEV)

        barrier_sem = pltpu.get_barrier_semaphore()
        for nbr in (left, right):
            pl.semaphore_signal(
                barrier_sem,
                inc=1,
                device_id=(nbr,),
                device_id_type=pl.DeviceIdType.MESH,
            )

        def chunk_f(c):
            xc = x_ref[pl.ds(c * m_chunk, m_chunk), :]
            return jnp.dot(
                xc, w_ref[:, :n_half], preferred_element_type=jnp.float32
            )

        def chunk_b(c):
            xc = x_ref[pl.ds(c * m_chunk, m_chunk), :]
            return jnp.dot(
                xc, w_ref[:, n_half:], preferred_element_type=jnp.float32
            )

        def md(v):
            return lax.rem(v + 2 * N_DEV, N_DEV)

        comm_f[N_DEV - 1] = chunk_f(md(d - 1)).astype(jnp.bfloat16)
        comm_b[N_DEV - 1] = chunk_b(md(d + 1)).astype(jnp.bfloat16)

        m_sub = m_chunk // N_SUB

        def mk(s, backward, j):
            comm = comm_b if backward else comm_f
            src_slot = (N_DEV - 1) if s == 0 else s - 1
            rows = slice(j * m_sub, (j + 1) * m_sub)
            return pltpu.make_async_remote_copy(
                src_ref=comm.at[src_slot, rows, :],
                dst_ref=comm.at[s, rows, :],
                send_sem=(send_b if backward else send_f).at[s, j],
                recv_sem=(recv_b if backward else recv_f).at[s, j],
                device_id=(left,) if backward else (right,),
                device_id_type=pl.DeviceIdType.MESH,
            )

        pl.semaphore_wait(barrier_sem, 2)

        rdmas = []
        for j in range(N_SUB):
            rdmas += [mk(0, False, j), mk(0, True, j)]
        for r in rdmas:
            r.start()

        pre_f = chunk_f(md(d - 2)).astype(jnp.bfloat16)
        pre_b = chunk_b(md(d + 2)).astype(jnp.bfloat16)

        scale = sx_ref[0] * sw_ref[0]

        def epilogue(acc):
            y = acc * scale
            z = jnp.clip(y, -60.0, 60.0)
            return y / (1.0 + jnp.exp(-z))

        n_pieces = 2 * N_SUB
        for s in range(N_DEV - 1):
            step = rdmas[n_pieces * s : n_pieces * (s + 1)]
            pieces = []
            for j in range(N_SUB):
                rows = slice(j * m_sub, (j + 1) * m_sub)
                pieces.append((step[2 * j], comm_f, rows, False, j))
                pieces.append((step[2 * j + 1], comm_b, rows, True, j))
            if s < N_DEV - 2:
                nxt = []
                for r, comm, rows, bwd, j in pieces:
                    nr = mk(s + 1, bwd, j)
                    nr.start()
                    nxt.append(nr)
                rdmas += nxt
            else:
                for r, comm, rows, bwd, j in pieces:
                    pre = pre_b if bwd else pre_f
                    r.wait_recv()
                    acc = comm[s, rows, :].astype(jnp.float32) + pre[
                        rows, :
                    ].astype(jnp.float32)
                    cols = slice(n_half, n) if bwd else slice(0, n_half)
                    out_ref[rows, cols] = epilogue(acc)

        for r in rdmas:
            r.wait_send()
        for r in rdmas[: -2 * N_SUB]:
            r.wait_recv()

    return pl.pallas_call(
        body,
        out_shape=jax.ShapeDtypeStruct((m_chunk, n), jnp.float32),
        in_specs=[
            pl.BlockSpec(memory_space=pltpu.VMEM),
            pl.BlockSpec(memory_space=pltpu.VMEM),
            pl.BlockSpec(memory_space=pltpu.SMEM),
            pl.BlockSpec(memory_space=pltpu.SMEM),
        ],
        out_specs=pl.BlockSpec(memory_space=pltpu.VMEM),
        scratch_shapes=[
            pltpu.VMEM((N_DEV, m_chunk, n_half), jnp.bfloat16),
            pltpu.VMEM((N_DEV, m_chunk, n_half), jnp.bfloat16),
            pltpu.SemaphoreType.DMA((N_DEV - 1, N_SUB)),
            pltpu.SemaphoreType.DMA((N_DEV - 1, N_SUB)),
            pltpu.SemaphoreType.DMA((N_DEV - 1, N_SUB)),
            pltpu.SemaphoreType.DMA((N_DEV - 1, N_SUB)),
        ],
        compiler_params=pltpu.CompilerParams(collective_id=0),
    )(x, w_mat, sx, sw)


# device time: 98468 ns/iter; 1.0009x vs baseline; 1.0009x over previous
import jax
import jax.numpy as jnp
from jax import lax
from jax.experimental import pallas as pl
from jax.experimental.pallas import tpu as pltpu

N_DEV = 16
N_SUB = 2


def kernel(x, w_mat, scale_x, scale_w):
    m, _ = x.shape
    _, n = w_mat.shape
    m_chunk = m // N_DEV
    n_half = n // 2

    sx = scale_x.reshape(1).astype(jnp.float32)
    sw = scale_w.reshape(1).astype(jnp.float32)

    def body(
        x_ref,
        w_ref,
        sx_ref,
        sw_ref,
        out_ref,
        comm_f,
        comm_b,
        send_f,
        recv_f,
        send_b,
        recv_b,
    ):
        d = lax.axis_index("i")
        right = lax.rem(d + 1, N_DEV)
        left = lax.rem(d - 1 + N_DEV, N_DEV)

        barrier_sem = pltpu.get_barrier_semaphore()
        for nbr in (left, right):
            pl.semaphore_signal(
                barrier_sem,
                inc=1,
                device_id=(nbr,),
                device_id_type=pl.DeviceIdType.MESH,
            )

        def chunk_f(c):
            xc = x_ref[pl.ds(c * m_chunk, m_chunk), :]
            return jnp.dot(
                xc, w_ref[:, :n_half], preferred_element_type=jnp.float32
            )

        def chunk_b(c):
            xc = x_ref[pl.ds(c * m_chunk, m_chunk), :]
            return jnp.dot(
                xc, w_ref[:, n_half:], preferred_element_type=jnp.float32
            )

        def md(v):
            return lax.rem(v + 2 * N_DEV, N_DEV)

        comm_f[N_DEV - 1] = chunk_f(md(d - 1)).astype(jnp.bfloat16)
        comm_b[N_DEV - 1] = chunk_b(md(d + 1)).astype(jnp.bfloat16)

        m_sub = m_chunk // N_SUB

        def mk(s, backward, j):
            comm = comm_b if backward else comm_f
            src_slot = (N_DEV - 1) if s == 0 else s - 1
            rows = slice(j * m_sub, (j + 1) * m_sub)
            return pltpu.make_async_remote_copy(
                src_ref=comm.at[src_slot, rows, :],
                dst_ref=comm.at[s, rows, :],
                send_sem=(send_b if backward else send_f).at[s, j],
                recv_sem=(recv_b if backward else recv_f).at[s, j],
                device_id=(left,) if backward else (right,),
                device_id_type=pl.DeviceIdType.MESH,
            )

        pl.semaphore_wait(barrier_sem, 2)

        rdmas = []
        for j in range(N_SUB):
            rdmas += [mk(0, False, j), mk(0, True, j)]
        for r in rdmas:
            r.start()

        pre_f = chunk_f(md(d - 2)).astype(jnp.bfloat16)
        pre_b = chunk_b(md(d + 2)).astype(jnp.bfloat16)

        scale = sx_ref[0] * sw_ref[0]

        def epilogue(acc):
            y = acc * scale
            z = jnp.clip(y, -60.0, 60.0)
            return y / (1.0 + jnp.exp(-z))

        n_pieces = 2 * N_SUB
        for s in range(N_DEV - 1):
            step = rdmas[n_pieces * s : n_pieces * (s + 1)]
            pieces = []
            for j in range(N_SUB):
                rows = slice(j * m_sub, (j + 1) * m_sub)
                pieces.append((step[2 * j], comm_f, rows, False, j))
                pieces.append((step[2 * j + 1], comm_b, rows, True, j))
            if s < N_DEV - 2:
                nxt = []
                for r, comm, rows, bwd, j in pieces:
                    pre = pre_b if bwd else pre_f
                    r.wait_recv()
                    comm[s, rows, :] = comm[s, rows, :] + pre[rows, :]
                    nr = mk(s + 1, bwd, j)
                    nr.start()
                    nxt.append(nr)
                rdmas += nxt
                pre_f = chunk_f(md(d - s - 3)).astype(jnp.bfloat16)
                pre_b = chunk_b(md(d + s + 3)).astype(jnp.bfloat16)
            else:
                for r, comm, rows, bwd, j in pieces:
                    pre = pre_b if bwd else pre_f
                    r.wait_recv()
                    acc = comm[s, rows, :].astype(jnp.float32) + pre[
                        rows, :
                    ].astype(jnp.float32)
                    cols = slice(n_half, n) if bwd else slice(0, n_half)
                    out_ref[rows, cols] = epilogue(acc)

        for r in rdmas:
            r.wait_send()

    return pl.pallas_call(
        body,
        out_shape=jax.ShapeDtypeStruct((m_chunk, n), jnp.float32),
        in_specs=[
            pl.BlockSpec(memory_space=pltpu.VMEM),
            pl.BlockSpec(memory_space=pltpu.VMEM),
            pl.BlockSpec(memory_space=pltpu.SMEM),
            pl.BlockSpec(memory_space=pltpu.SMEM),
        ],
        out_specs=pl.BlockSpec(memory_space=pltpu.VMEM),
        scratch_shapes=[
            pltpu.VMEM((N_DEV, m_chunk, n_half), jnp.bfloat16),
            pltpu.VMEM((N_DEV, m_chunk, n_half), jnp.bfloat16),
            pltpu.SemaphoreType.DMA((N_DEV - 1, N_SUB)),
            pltpu.SemaphoreType.DMA((N_DEV - 1, N_SUB)),
            pltpu.SemaphoreType.DMA((N_DEV - 1, N_SUB)),
            pltpu.SemaphoreType.DMA((N_DEV - 1, N_SUB)),
        ],
        compiler_params=pltpu.CompilerParams(collective_id=0),
    )(x, w_mat, sx, sw)


# device time: 95598 ns/iter; 1.0309x vs baseline; 1.0300x over previous
import jax
import jax.numpy as jnp
from jax import lax
from jax.experimental import pallas as pl
from jax.experimental.pallas import tpu as pltpu

N_DEV = 16
N_SUB = 2

PERM = [0, 1, 5, 9, 13, 14, 10, 6, 2, 3, 7, 11, 15, 12, 8, 4]
POS = {dev: i for i, dev in enumerate(PERM)}
NEXT = {PERM[i]: PERM[(i + 1) % N_DEV] for i in range(N_DEV)}
PREV = {PERM[i]: PERM[(i - 1) % N_DEV] for i in range(N_DEV)}


def kernel(x, w_mat, scale_x, scale_w):
    m, _ = x.shape
    _, n = w_mat.shape
    m_chunk = m // N_DEV
    n_half = n // 2

    sx = scale_x.reshape(1).astype(jnp.float32)
    sw = scale_w.reshape(1).astype(jnp.float32)

    def body(
        x_ref,
        w_ref,
        sx_ref,
        sw_ref,
        out_ref,
        comm_f,
        comm_b,
        send_f,
        recv_f,
        send_b,
        recv_b,
    ):
        d = lax.axis_index("i")

        def sel(tbl, v):
            out = jnp.int32(tbl[N_DEV - 1])
            for k in range(N_DEV - 2, -1, -1):
                out = jnp.where(v == k, jnp.int32(tbl[k]), out)
            return out

        right = sel([NEXT[k] for k in range(N_DEV)], d)
        left = sel([PREV[k] for k in range(N_DEV)], d)
        pos = sel([POS[k] for k in range(N_DEV)], d)

        barrier_sem = pltpu.get_barrier_semaphore()
        for nbr in (left, right):
            pl.semaphore_signal(
                barrier_sem,
                inc=1,
                device_id=(nbr,),
                device_id_type=pl.DeviceIdType.MESH,
            )

        def chunk_f(c):
            xc = x_ref[pl.ds(c * m_chunk, m_chunk), :]
            return jnp.dot(
                xc, w_ref[:, :n_half], preferred_element_type=jnp.float32
            )

        def chunk_b(c):
            xc = x_ref[pl.ds(c * m_chunk, m_chunk), :]
            return jnp.dot(
                xc, w_ref[:, n_half:], preferred_element_type=jnp.float32
            )

        def md(v):
            return lax.rem(v + 2 * N_DEV, N_DEV)

        def chunk_at(p):
            return sel(PERM, md(p))

        comm_f[N_DEV - 1] = chunk_f(chunk_at(pos - 1)).astype(jnp.bfloat16)
        comm_b[N_DEV - 1] = chunk_b(chunk_at(pos + 1)).astype(jnp.bfloat16)

        m_sub = m_chunk // N_SUB

        def mk(s, backward, j):
            comm = comm_b if backward else comm_f
            src_slot = (N_DEV - 1) if s == 0 else s - 1
            rows = slice(j * m_sub, (j + 1) * m_sub)
            return pltpu.make_async_remote_copy(
                src_ref=comm.at[src_slot, rows, :],
                dst_ref=comm.at[s, rows, :],
                send_sem=(send_b if backward else send_f).at[s, j],
                recv_sem=(recv_b if backward else recv_f).at[s, j],
                device_id=(left,) if backward else (right,),
                device_id_type=pl.DeviceIdType.MESH,
            )

        pl.semaphore_wait(barrier_sem, 2)

        rdmas = []
        for j in range(N_SUB):
            rdmas += [mk(0, False, j), mk(0, True, j)]
        for r in rdmas:
            r.start()

        pre_f = chunk_f(chunk_at(pos - 2)).astype(jnp.bfloat16)
        pre_b = chunk_b(chunk_at(pos + 2)).astype(jnp.bfloat16)

        scale = sx_ref[0] * sw_ref[0]

        def epilogue(acc):
            y = acc * scale
            z = jnp.clip(y, -60.0, 60.0)
            return y / (1.0 + jnp.exp(-z))

        n_pieces = 2 * N_SUB
        for s in range(N_DEV - 1):
            step = rdmas[n_pieces * s : n_pieces * (s + 1)]
            pieces = []
            for j in range(N_SUB):
                rows = slice(j * m_sub, (j + 1) * m_sub)
                pieces.append((step[2 * j], comm_f, rows, False, j))
                pieces.append((step[2 * j + 1], comm_b, rows, True, j))
            if s < N_DEV - 2:
                nxt = []
                for r, comm, rows, bwd, j in pieces:
                    pre = pre_b if bwd else pre_f
                    r.wait_recv()
                    comm[s, rows, :] = comm[s, rows, :] + pre[rows, :]
                    nr = mk(s + 1, bwd, j)
                    nr.start()
                    nxt.append(nr)
                rdmas += nxt
                pre_f = chunk_f(chunk_at(pos - s - 3)).astype(jnp.bfloat16)
                pre_b = chunk_b(chunk_at(pos + s + 3)).astype(jnp.bfloat16)
            else:
                for r, comm, rows, bwd, j in pieces:
                    pre = pre_b if bwd else pre_f
                    r.wait_recv()
                    acc = comm[s, rows, :].astype(jnp.float32) + pre[
                        rows, :
                    ].astype(jnp.float32)
                    cols = slice(n_half, n) if bwd else slice(0, n_half)
                    out_ref[rows, cols] = epilogue(acc)

        for r in rdmas:
            r.wait_send()

    return pl.pallas_call(
        body,
        out_shape=jax.ShapeDtypeStruct((m_chunk, n), jnp.float32),
        in_specs=[
            pl.BlockSpec(memory_space=pltpu.VMEM),
            pl.BlockSpec(memory_space=pltpu.VMEM),
            pl.BlockSpec(memory_space=pltpu.SMEM),
            pl.BlockSpec(memory_space=pltpu.SMEM),
        ],
        out_specs=pl.BlockSpec(memory_space=pltpu.VMEM),
        scratch_shapes=[
            pltpu.VMEM((N_DEV, m_chunk, n_half), jnp.bfloat16),
            pltpu.VMEM((N_DEV, m_chunk, n_half), jnp.bfloat16),
            pltpu.SemaphoreType.DMA((N_DEV - 1, N_SUB)),
            pltpu.SemaphoreType.DMA((N_DEV - 1, N_SUB)),
            pltpu.SemaphoreType.DMA((N_DEV - 1, N_SUB)),
            pltpu.SemaphoreType.DMA((N_DEV - 1, N_SUB)),
        ],
        compiler_params=pltpu.CompilerParams(collective_id=0),
    )(x, w_mat, sx, sw)
